# Initial kernel scaffold; baseline (speedup 1.0000x reference)
import jax
import jax.numpy as jnp
from jax.experimental import pallas as pl


def kernel(ids, pads, emb_table):
    raise NotImplementedError("write your pallas kernel here")


# speedup vs baseline: 6.9049x; 6.9049x over previous
import functools

import jax
import jax.numpy as jnp
from jax import lax
from jax.experimental import pallas as pl
from jax.experimental.pallas import tpu as pltpu
from jax.experimental.pallas import tpu_sc as plsc

_VOCAB = 100000
_N_EMB = _VOCAB + 2
_DIM = 128
_B = 4096
_L = 200
_PADDING_ID = _N_EMB - 1

_CH = 50
_NCH = _L // _CH
_NV = _DIM // 16

_NWORKERS = 32
_ROWS_PER_W = _B // _NWORKERS


def _pooler_body(ids_hbm, pads_hbm, table_hbm, out_hbm,
                 ids_v, pads_v, padvec_v, buf_v, out_v, sem):
    nc = 2
    wid = lax.axis_index("s") * nc + lax.axis_index("c")
    base = wid * _ROWS_PER_W

    pltpu.sync_copy(ids_hbm.at[pl.ds(base * _NCH, _ROWS_PER_W * _NCH)], ids_v)
    pltpu.sync_copy(pads_hbm.at[pl.ds(base, _ROWS_PER_W)], pads_v)
    pltpu.sync_copy(table_hbm.at[pl.ds(_PADDING_ID, 1)], padvec_v)

    neg_inf = jnp.full((16,), -jnp.inf, jnp.float32)
    zero = jnp.zeros((16,), jnp.float32)
    lane = lax.iota(jnp.int32, 16)

    def row_body(r, carry):
        g = lax.shift_right_logical(r, 4)
        l = r - g * 16
        pvec = pads_v[pl.ds(g * 16, 16)]
        pads_r = jnp.max(jnp.where(lane == l, pvec, 0))
        n = _L - pads_r
        nchunks = (1 + (n > _CH).astype(jnp.int32)
                   + (n > 2 * _CH).astype(jnp.int32)
                   + (n > 3 * _CH).astype(jnp.int32))

        pv = tuple(padvec_v[0, pl.ds(16 * v, 16)] for v in range(_NV))
        init_max = tuple(
            jnp.where(pads_r > 0, pv[v], neg_inf) for v in range(_NV))
        init_sum = (zero,) * _NV

        def chunk_body(c, acc):
            idx_row = r * _NCH + c
            pltpu.async_copy(table_hbm.at[ids_v.at[idx_row]], buf_v, sem).wait()
            valid = jnp.minimum(n - c * _CH, _CH)

            def j_body(j, acc2):
                sums, maxs = acc2
                row = tuple(buf_v[j, pl.ds(16 * v, 16)] for v in range(_NV))
                return (tuple(sums[v] + row[v] for v in range(_NV)),
                        tuple(jnp.maximum(maxs[v], row[v]) for v in range(_NV)))

            return lax.fori_loop(0, valid, j_body, acc)

        sums, maxs = lax.fori_loop(0, nchunks, chunk_body,
                                   (init_sum, init_max))
        n_vec = jnp.full((16,), 1.0, jnp.float32) * n.astype(jnp.float32)
        inv = jnp.full((16,), 1.0, jnp.float32) / n_vec
        for v in range(_NV):
            out_v[r, pl.ds(16 * v, 16)] = maxs[v]
            out_v[r, pl.ds(_DIM + 16 * v, 16)] = sums[v] * inv
        return carry

    lax.fori_loop(0, _ROWS_PER_W, row_body, 0)
    pltpu.sync_copy(out_v, out_hbm.at[pl.ds(base, _ROWS_PER_W)])


@jax.jit
def _pooler(ids2, pads, emb_table):
    mesh = plsc.VectorSubcoreMesh(core_axis_name="c", subcore_axis_name="s")
    run = pl.kernel(
        _pooler_body,
        out_type=jax.ShapeDtypeStruct((_B, 2 * _DIM), jnp.float32),
        mesh=mesh,
        scratch_types=[
            pltpu.VMEM((_ROWS_PER_W * _NCH, _CH), jnp.int32),
            pltpu.VMEM((_ROWS_PER_W,), jnp.int32),
            pltpu.VMEM((1, _DIM), jnp.float32),
            pltpu.VMEM((_CH, _DIM), jnp.float32),
            pltpu.VMEM((_ROWS_PER_W, 2 * _DIM), jnp.float32),
            pltpu.SemaphoreType.DMA,
        ],
        compiler_params=pltpu.CompilerParams(needs_layout_passes=False),
    )
    return run(ids2, pads, emb_table)


def kernel(ids, pads, emb_table):
    ids2 = ids.reshape(_B * _NCH, _CH)
    return _pooler(ids2, pads, emb_table)


# speedup vs baseline: 15.5922x; 2.2581x over previous
import functools

import jax
import jax.numpy as jnp
from jax import lax
from jax.experimental import pallas as pl
from jax.experimental.pallas import tpu as pltpu
from jax.experimental.pallas import tpu_sc as plsc

_VOCAB = 100000
_N_EMB = _VOCAB + 2
_DIM = 128
_B = 4096
_L = 200
_PADDING_ID = _N_EMB - 1

_CH = 50
_NCH = _L // _CH
_NV = _DIM // 16

_NWORKERS = 32
_ROWS_PER_W = _B // _NWORKERS


def _pooler_body(ids_hbm, pads_hbm, table_hbm, out_hbm,
                 ids_v, pads_v, padvec_v, buf0_v, buf1_v, outrow_v,
                 sem0, sem1, semw0, semw1):
    nc = 2
    wid = lax.axis_index("s") * nc + lax.axis_index("c")
    base = wid * _ROWS_PER_W

    pltpu.sync_copy(ids_hbm.at[pl.ds(base * _NCH, _ROWS_PER_W * _NCH)], ids_v)
    pltpu.sync_copy(pads_hbm.at[pl.ds(base, _ROWS_PER_W)], pads_v)
    pltpu.sync_copy(table_hbm.at[pl.ds(_PADDING_ID, 1)], padvec_v)

    neg_inf = jnp.full((16,), -jnp.inf, jnp.float32)
    zero = jnp.zeros((16,), jnp.float32)
    lane = lax.iota(jnp.int32, 16)

    def seq_len(r):
        g = lax.shift_right_logical(r, 4)
        l = r - g * 16
        pvec = pads_v[pl.ds(g * 16, 16)]
        pads_r = jnp.max(jnp.where(lane == l, pvec, 0))
        return _L - pads_r

    def nchunks_of(n):
        return (1 + (n > _CH).astype(jnp.int32)
                + (n > 2 * _CH).astype(jnp.int32)
                + (n > 3 * _CH).astype(jnp.int32))

    def fire_row(r, buf, sem):
        nchunks = nchunks_of(seq_len(r))
        for c in range(_NCH):
            @pl.when(c < nchunks)
            def _():
                pltpu.async_copy(table_hbm.at[ids_v.at[r * _NCH + c]],
                                 buf.at[pl.ds(c * _CH, _CH)], sem)

    def compute_row(r, buf, sem, p, semw):
        n = seq_len(r)
        nchunks = nchunks_of(n)
        for c in range(_NCH):
            @pl.when(c < nchunks)
            def _():
                pltpu.make_async_copy(table_hbm.at[ids_v.at[r * _NCH + c]],
                                      buf.at[pl.ds(c * _CH, _CH)], sem).wait()

        pads_r = _L - n
        pv = tuple(padvec_v[0, pl.ds(16 * v, 16)] for v in range(_NV))
        acc = ((zero,) * _NV,
               tuple(jnp.where(pads_r > 0, pv[v], neg_inf)
                     for v in range(_NV)))

        def j_body(off):
            def body(j, acc2):
                sums, maxs = acc2
                row = tuple(buf[off + j, pl.ds(16 * v, 16)]
                            for v in range(_NV))
                return (tuple(sums[v] + row[v] for v in range(_NV)),
                        tuple(jnp.maximum(maxs[v], row[v])
                              for v in range(_NV)))
            return body

        for c in range(_NCH):
            valid = jnp.clip(n - c * _CH, 0, _CH)
            acc = lax.fori_loop(0, valid, j_body(c * _CH), acc)

        sums, maxs = acc
        n_vec = jnp.full((16,), 1.0, jnp.float32) * n.astype(jnp.float32)
        inv = jnp.full((16,), 1.0, jnp.float32) / n_vec
        @pl.when(r >= 2)
        def _():
            pltpu.make_async_copy(outrow_v.at[p], out_hbm.at[base + r - 2],
                                  semw).wait()
        for v in range(_NV):
            outrow_v[p, pl.ds(16 * v, 16)] = maxs[v]
            outrow_v[p, pl.ds(_DIM + 16 * v, 16)] = sums[v] * inv
        pltpu.async_copy(outrow_v.at[p], out_hbm.at[base + r], semw)

    fire_row(jnp.int32(0), buf0_v, sem0)

    def pair_body(rp, carry):
        r0 = rp * 2
        fire_row(r0 + 1, buf1_v, sem1)
        compute_row(r0, buf0_v, sem0, 0, semw0)

        @pl.when(r0 + 2 < _ROWS_PER_W)
        def _():
            fire_row(r0 + 2, buf0_v, sem0)
        compute_row(r0 + 1, buf1_v, sem1, 1, semw1)
        return carry

    lax.fori_loop(0, _ROWS_PER_W // 2, pair_body, 0)
    pltpu.make_async_copy(outrow_v.at[0],
                          out_hbm.at[base + _ROWS_PER_W - 2], semw0).wait()
    pltpu.make_async_copy(outrow_v.at[1],
                          out_hbm.at[base + _ROWS_PER_W - 1], semw1).wait()


@jax.jit
def _pooler(ids2, pads, emb_table):
    mesh = plsc.VectorSubcoreMesh(core_axis_name="c", subcore_axis_name="s")
    run = pl.kernel(
        _pooler_body,
        out_type=jax.ShapeDtypeStruct((_B, 2 * _DIM), jnp.float32),
        mesh=mesh,
        scratch_types=[
            pltpu.VMEM((_ROWS_PER_W * _NCH, _CH), jnp.int32),
            pltpu.VMEM((_ROWS_PER_W,), jnp.int32),
            pltpu.VMEM((1, _DIM), jnp.float32),
            pltpu.VMEM((_L, _DIM), jnp.float32),
            pltpu.VMEM((_L, _DIM), jnp.float32),
            pltpu.VMEM((2, 2 * _DIM), jnp.float32),
            pltpu.SemaphoreType.DMA,
            pltpu.SemaphoreType.DMA,
            pltpu.SemaphoreType.DMA,
            pltpu.SemaphoreType.DMA,
        ],
        compiler_params=pltpu.CompilerParams(needs_layout_passes=False),
    )
    return run(ids2, pads, emb_table)


def kernel(ids, pads, emb_table):
    ids2 = ids.reshape(_B * _NCH, _CH)
    return _pooler(ids2, pads, emb_table)


# speedup vs baseline: 15.6840x; 1.0059x over previous
import functools

import jax
import jax.numpy as jnp
from jax import lax
from jax.experimental import pallas as pl
from jax.experimental.pallas import tpu as pltpu
from jax.experimental.pallas import tpu_sc as plsc

_VOCAB = 100000
_N_EMB = _VOCAB + 2
_DIM = 128
_B = 4096
_L = 200
_PADDING_ID = _N_EMB - 1

_CH = 50
_NCH = _L // _CH
_NV = _DIM // 16

_NWORKERS = 32
_ROWS_PER_W = _B // _NWORKERS


def _pooler_body(ids_hbm, pads_hbm, table_hbm, out_hbm,
                 ids_v, pads_v, padvec_v, buf0_v, buf1_v, outrow_v,
                 sem0, sem1, semw0, semw1):
    nc = 2
    wid = lax.axis_index("s") * nc + lax.axis_index("c")
    base = wid * _ROWS_PER_W

    pltpu.sync_copy(ids_hbm.at[pl.ds(base * _NCH, _ROWS_PER_W * _NCH)], ids_v)
    pltpu.sync_copy(pads_hbm.at[pl.ds(base, _ROWS_PER_W)], pads_v)
    pltpu.sync_copy(table_hbm.at[pl.ds(_PADDING_ID, 1)], padvec_v)

    neg_inf = jnp.full((16,), -jnp.inf, jnp.float32)
    zero = jnp.zeros((16,), jnp.float32)
    lane = lax.iota(jnp.int32, 16)

    def seq_len(r):
        g = lax.shift_right_logical(r, 4)
        l = r - g * 16
        pvec = pads_v[pl.ds(g * 16, 16)]
        pads_r = jnp.max(jnp.where(lane == l, pvec, 0))
        return _L - pads_r

    def nchunks_of(n):
        return (1 + (n > _CH).astype(jnp.int32)
                + (n > 2 * _CH).astype(jnp.int32)
                + (n > 3 * _CH).astype(jnp.int32))

    def fire_row(r, buf, sem):
        nchunks = nchunks_of(seq_len(r))
        for c in range(_NCH):
            @pl.when(c < nchunks)
            def _():
                pltpu.async_copy(table_hbm.at[ids_v.at[r * _NCH + c]],
                                 buf.at[pl.ds(c * _CH, _CH)], sem)

    def compute_row(r, buf, sem, p, semw):
        n = seq_len(r)
        nchunks = nchunks_of(n)
        for c in range(_NCH):
            @pl.when(c < nchunks)
            def _():
                pltpu.make_async_copy(table_hbm.at[ids_v.at[r * _NCH + c]],
                                      buf.at[pl.ds(c * _CH, _CH)], sem).wait()

        pads_r = _L - n
        pv = tuple(padvec_v[0, pl.ds(16 * v, 16)] for v in range(_NV))
        acc = ((zero,) * _NV,
               tuple(jnp.where(pads_r > 0, pv[v], neg_inf)
                     for v in range(_NV)))

        def body(j, acc2):
            sums, maxs = acc2
            row = tuple(buf[j, pl.ds(16 * v, 16)] for v in range(_NV))
            return (tuple(sums[v] + row[v] for v in range(_NV)),
                    tuple(jnp.maximum(maxs[v], row[v])
                          for v in range(_NV)))

        sums, maxs = plsc.parallel_loop(0, n, 1, unroll=4, carry=acc)(body)
        n_vec = jnp.full((16,), 1.0, jnp.float32) * n.astype(jnp.float32)
        inv = jnp.full((16,), 1.0, jnp.float32) / n_vec
        @pl.when(r >= 2)
        def _():
            pltpu.make_async_copy(outrow_v.at[p], out_hbm.at[base + r - 2],
                                  semw).wait()
        for v in range(_NV):
            outrow_v[p, pl.ds(16 * v, 16)] = maxs[v]
            outrow_v[p, pl.ds(_DIM + 16 * v, 16)] = sums[v] * inv
        pltpu.async_copy(outrow_v.at[p], out_hbm.at[base + r], semw)

    fire_row(jnp.int32(0), buf0_v, sem0)

    def pair_body(rp, carry):
        r0 = rp * 2
        fire_row(r0 + 1, buf1_v, sem1)
        compute_row(r0, buf0_v, sem0, 0, semw0)

        @pl.when(r0 + 2 < _ROWS_PER_W)
        def _():
            fire_row(r0 + 2, buf0_v, sem0)
        compute_row(r0 + 1, buf1_v, sem1, 1, semw1)
        return carry

    lax.fori_loop(0, _ROWS_PER_W // 2, pair_body, 0)
    pltpu.make_async_copy(outrow_v.at[0],
                          out_hbm.at[base + _ROWS_PER_W - 2], semw0).wait()
    pltpu.make_async_copy(outrow_v.at[1],
                          out_hbm.at[base + _ROWS_PER_W - 1], semw1).wait()


@jax.jit
def _pooler(ids2, pads, emb_table):
    mesh = plsc.VectorSubcoreMesh(core_axis_name="c", subcore_axis_name="s")
    run = pl.kernel(
        _pooler_body,
        out_type=jax.ShapeDtypeStruct((_B, 2 * _DIM), jnp.float32),
        mesh=mesh,
        scratch_types=[
            pltpu.VMEM((_ROWS_PER_W * _NCH, _CH), jnp.int32),
            pltpu.VMEM((_ROWS_PER_W,), jnp.int32),
            pltpu.VMEM((1, _DIM), jnp.float32),
            pltpu.VMEM((_L, _DIM), jnp.float32),
            pltpu.VMEM((_L, _DIM), jnp.float32),
            pltpu.VMEM((2, 2 * _DIM), jnp.float32),
            pltpu.SemaphoreType.DMA,
            pltpu.SemaphoreType.DMA,
            pltpu.SemaphoreType.DMA,
            pltpu.SemaphoreType.DMA,
        ],
        compiler_params=pltpu.CompilerParams(needs_layout_passes=False),
    )
    return run(ids2, pads, emb_table)


def kernel(ids, pads, emb_table):
    ids2 = ids.reshape(_B * _NCH, _CH)
    return _pooler(ids2, pads, emb_table)
